# baseline (speedup 1.0000x reference)
import jax
import jax.numpy as jnp
from jax import lax
from jax.experimental import pallas as pl
from jax.experimental.pallas import tpu as pltpu
from jax.experimental.pallas import tpu_sc as plsc

EMBED_DIM = 64
BATCH = 16384
N_ROWS = 1000001
LANES = 128

_info = plsc.get_sparse_core_info()
NC, NS, L = _info.num_cores, _info.num_subcores, _info.num_lanes
NW = NC * NS
B_PER_W = BATCH // NW
EPB = 2
SLOTS = 2


def _mf_kernel(user_hbm, item_hbm, uemb_hbm, iemb_hbm, out_hbm,
               idx_u, idx_i, buf_u, buf_i, out_v, sem0, sem1):
    wid = lax.axis_index("s") * NC + lax.axis_index("c")
    base = wid * B_PER_W

    pltpu.sync_copy(user_hbm.at[pl.ds(base, B_PER_W)], idx_u)
    pltpu.sync_copy(item_hbm.at[pl.ds(base, B_PER_W)], idx_i)

    lane = lax.iota(jnp.int32, L)
    dvs = [j * L + lane for j in range(EMBED_DIM // L)]
    sems = [sem0, sem1]

    NSB = L // EPB
    NGROUPS = B_PER_W // L

    def window(emb, r):
        return emb.at[:, pl.ds((r // LANES) * LANES, LANES)]

    def issue(uvec, ivec, sb, slot):
        copies = []
        lanes_u = []
        lanes_i = []
        for e in range(EPB):
            ru = uvec[sb * EPB + e]
            ri = ivec[sb * EPB + e]
            copies.append(pltpu.async_copy(
                window(uemb_hbm, ru), buf_u.at[slot * EPB + e], sems[slot]))
            copies.append(pltpu.async_copy(
                window(iemb_hbm, ri), buf_i.at[slot * EPB + e], sems[slot]))
            lanes_u.append(jnp.full((L,), ru % LANES, jnp.int32))
            lanes_i.append(jnp.full((L,), ri % LANES, jnp.int32))
        return copies, lanes_u, lanes_i

    def group_body(g, carry):
        base_g = g * L
        uvec = idx_u[pl.ds(base_g, L)]
        ivec = idx_i[pl.ds(base_g, L)]
        res = jnp.zeros((L,), jnp.float32)
        pending = issue(uvec, ivec, 0, 0)
        for sb in range(NSB):
            slot = sb % SLOTS
            copies, lanes_u, lanes_i = pending
            if sb + 1 < NSB:
                pending = issue(uvec, ivec, sb + 1, (sb + 1) % SLOTS)
            for c in copies:
                c.wait()
            for e in range(EPB):
                acc = jnp.zeros((L,), jnp.float32)
                for j in range(EMBED_DIM // L):
                    gu = plsc.load_gather(
                        buf_u.at[slot * EPB + e], [dvs[j], lanes_u[e]])
                    gi = plsc.load_gather(
                        buf_i.at[slot * EPB + e], [dvs[j], lanes_i[e]])
                    acc = acc + gu * gi
                res = jnp.where(lane == sb * EPB + e, jnp.sum(acc), res)
        out_v[pl.ds(base_g, L)] = 1.0 / (1.0 + jnp.exp(-res))
        return carry

    lax.fori_loop(0, NGROUPS, group_body, 0)

    pltpu.sync_copy(out_v, out_hbm.at[pl.ds(base, B_PER_W)])


@jax.jit
def kernel(user, item, user_embed, item_embed):
    user = user.astype(jnp.int32)
    item = item.astype(jnp.int32)
    uemb_t = user_embed.T
    iemb_t = item_embed.T
    mesh = plsc.VectorSubcoreMesh(core_axis_name="c", subcore_axis_name="s")
    f = pl.kernel(
        _mf_kernel,
        mesh=mesh,
        compiler_params=pltpu.CompilerParams(needs_layout_passes=False),
        out_type=jax.ShapeDtypeStruct((BATCH,), jnp.float32),
        scratch_types=[
            pltpu.VMEM((B_PER_W,), jnp.int32),
            pltpu.VMEM((B_PER_W,), jnp.int32),
            pltpu.VMEM((SLOTS * EPB, EMBED_DIM, LANES), jnp.float32),
            pltpu.VMEM((SLOTS * EPB, EMBED_DIM, LANES), jnp.float32),
            pltpu.VMEM((B_PER_W,), jnp.float32),
            pltpu.SemaphoreType.DMA,
            pltpu.SemaphoreType.DMA,
        ],
    )
    return f(user, item, uemb_t, iemb_t)


# speedup vs baseline: 1.0210x; 1.0210x over previous
import jax
import jax.numpy as jnp
from jax import lax
from jax.experimental import pallas as pl
from jax.experimental.pallas import tpu as pltpu
from jax.experimental.pallas import tpu_sc as plsc

EMBED_DIM = 64
BATCH = 16384
N_ROWS = 1000001
LANES = 128

_info = plsc.get_sparse_core_info()
NC, NS, L = _info.num_cores, _info.num_subcores, _info.num_lanes
NW = NC * NS
B_PER_W = BATCH // NW
EPB = 2
SLOTS = 3


def _mf_kernel(user_hbm, item_hbm, uemb_hbm, iemb_hbm, out_hbm,
               idx_u, idx_i, buf_u, buf_i, out_v, sem0, sem1, sem2):
    wid = lax.axis_index("s") * NC + lax.axis_index("c")
    base = wid * B_PER_W

    pltpu.sync_copy(user_hbm.at[pl.ds(base, B_PER_W)], idx_u)
    pltpu.sync_copy(item_hbm.at[pl.ds(base, B_PER_W)], idx_i)

    lane = lax.iota(jnp.int32, L)
    dvs = [j * L + lane for j in range(EMBED_DIM // L)]
    sems = [sem0, sem1, sem2]

    NSB = L // EPB
    NGROUPS = B_PER_W // L

    def window(emb, r):
        return emb.at[:, pl.ds((r // LANES) * LANES, LANES)]

    def issue(uvec, ivec, sb, slot):
        copies = []
        lanes_u = []
        lanes_i = []
        for e in range(EPB):
            ru = uvec[sb * EPB + e]
            ri = ivec[sb * EPB + e]
            copies.append(pltpu.async_copy(
                window(uemb_hbm, ru), buf_u.at[slot * EPB + e], sems[slot]))
            copies.append(pltpu.async_copy(
                window(iemb_hbm, ri), buf_i.at[slot * EPB + e], sems[slot]))
            lanes_u.append(jnp.full((L,), ru % LANES, jnp.int32))
            lanes_i.append(jnp.full((L,), ri % LANES, jnp.int32))
        return copies, lanes_u, lanes_i

    def group_body(g, carry):
        base_g = g * L
        uvec = idx_u[pl.ds(base_g, L)]
        ivec = idx_i[pl.ds(base_g, L)]
        res = jnp.zeros((L,), jnp.float32)
        pending = {0: issue(uvec, ivec, 0, 0), 1: issue(uvec, ivec, 1, 1)}
        for sb in range(NSB):
            slot = sb % SLOTS
            copies, lanes_u, lanes_i = pending.pop(sb)
            if sb + 2 < NSB:
                pending[sb + 2] = issue(uvec, ivec, sb + 2, (sb + 2) % SLOTS)
            for c in copies:
                c.wait()
            for e in range(EPB):
                acc = jnp.zeros((L,), jnp.float32)
                for j in range(EMBED_DIM // L):
                    gu = plsc.load_gather(
                        buf_u.at[slot * EPB + e], [dvs[j], lanes_u[e]])
                    gi = plsc.load_gather(
                        buf_i.at[slot * EPB + e], [dvs[j], lanes_i[e]])
                    acc = acc + gu * gi
                res = jnp.where(lane == sb * EPB + e, jnp.sum(acc), res)
        out_v[pl.ds(base_g, L)] = 1.0 / (1.0 + jnp.exp(-res))
        return carry

    lax.fori_loop(0, NGROUPS, group_body, 0)

    pltpu.sync_copy(out_v, out_hbm.at[pl.ds(base, B_PER_W)])


@jax.jit
def kernel(user, item, user_embed, item_embed):
    user = user.astype(jnp.int32)
    item = item.astype(jnp.int32)
    uemb_t = user_embed.T
    iemb_t = item_embed.T
    mesh = plsc.VectorSubcoreMesh(core_axis_name="c", subcore_axis_name="s")
    f = pl.kernel(
        _mf_kernel,
        mesh=mesh,
        compiler_params=pltpu.CompilerParams(needs_layout_passes=False),
        out_type=jax.ShapeDtypeStruct((BATCH,), jnp.float32),
        scratch_types=[
            pltpu.VMEM((B_PER_W,), jnp.int32),
            pltpu.VMEM((B_PER_W,), jnp.int32),
            pltpu.VMEM((SLOTS * EPB, EMBED_DIM, LANES), jnp.float32),
            pltpu.VMEM((SLOTS * EPB, EMBED_DIM, LANES), jnp.float32),
            pltpu.VMEM((B_PER_W,), jnp.float32),
            pltpu.SemaphoreType.DMA,
            pltpu.SemaphoreType.DMA,
            pltpu.SemaphoreType.DMA,
        ],
    )
    return f(user, item, uemb_t, iemb_t)


# speedup vs baseline: 1.1915x; 1.1670x over previous
import jax
import jax.numpy as jnp
from jax import lax
from jax.experimental import pallas as pl
from jax.experimental.pallas import tpu as pltpu
from jax.experimental.pallas import tpu_sc as plsc

EMBED_DIM = 64
BATCH = 16384
N_ROWS = 1000001
LANES = 128

_info = plsc.get_sparse_core_info()
NC, NS, L = _info.num_cores, _info.num_subcores, _info.num_lanes
NW = NC * NS
B_PER_W = BATCH // NW
EPB = 2
SLOTS = 3


def _mf_kernel(user_hbm, item_hbm, uemb_hbm, iemb_hbm, out_hbm,
               idx_u, idx_i, buf_u, buf_i, out_v, sem0, sem1, sem2):
    wid = lax.axis_index("s") * NC + lax.axis_index("c")
    base = wid * B_PER_W

    pltpu.sync_copy(user_hbm.at[pl.ds(base, B_PER_W)], idx_u)
    pltpu.sync_copy(item_hbm.at[pl.ds(base, B_PER_W)], idx_i)

    lane = lax.iota(jnp.int32, L)
    dvs = [j * L + lane for j in range(EMBED_DIM // L)]
    sems = [sem0, sem0, sem0]

    NSB = L // EPB
    NGROUPS = B_PER_W // L

    def window(emb, r):
        return emb.at[:, pl.ds((r // LANES) * LANES, LANES)]

    def issue(uvec, ivec, sb, slot):
        copies = []
        lanes_u = []
        lanes_i = []
        for e in range(EPB):
            ru = uvec[sb * EPB + e]
            ri = ivec[sb * EPB + e]
            copies.append(pltpu.async_copy(
                window(uemb_hbm, ru), buf_u.at[slot * EPB + e], sems[slot]))
            copies.append(pltpu.async_copy(
                window(iemb_hbm, ri), buf_i.at[slot * EPB + e], sems[slot]))
            lanes_u.append(jnp.full((L,), ru % LANES, jnp.int32))
            lanes_i.append(jnp.full((L,), ri % LANES, jnp.int32))
        return copies, lanes_u, lanes_i

    def group_body(g, carry):
        base_g = g * L
        uvec = idx_u[pl.ds(base_g, L)]
        ivec = idx_i[pl.ds(base_g, L)]
        res = jnp.zeros((L,), jnp.float32)
        pending = {0: issue(uvec, ivec, 0, 0), 1: issue(uvec, ivec, 1, 1)}
        for sb in range(NSB):
            slot = sb % SLOTS
            copies, lanes_u, lanes_i = pending.pop(sb)
            if sb + 2 < NSB:
                pending[sb + 2] = issue(uvec, ivec, sb + 2, (sb + 2) % SLOTS)
            for c in copies:
                c.wait()
            for e in range(EPB):
                acc = jnp.zeros((L,), jnp.float32)
                for j in range(EMBED_DIM // L):
                    gu = plsc.load_gather(
                        buf_u.at[slot * EPB + e], [dvs[j], lanes_u[e]])
                    gi = plsc.load_gather(
                        buf_i.at[slot * EPB + e], [dvs[j], lanes_i[e]])
                    acc = acc + gu * gi
                res = jnp.where(lane == sb * EPB + e, jnp.sum(acc), res)
        out_v[pl.ds(base_g, L)] = 1.0 / (1.0 + jnp.exp(-res))
        return carry

    lax.fori_loop(0, NGROUPS, group_body, 0)

    pltpu.sync_copy(out_v, out_hbm.at[pl.ds(base, B_PER_W)])


@jax.jit
def kernel(user, item, user_embed, item_embed):
    user = user.astype(jnp.int32)
    item = item.astype(jnp.int32)
    uemb_t = user_embed.T
    iemb_t = item_embed.T
    mesh = plsc.VectorSubcoreMesh(core_axis_name="c", subcore_axis_name="s")
    f = pl.kernel(
        _mf_kernel,
        mesh=mesh,
        compiler_params=pltpu.CompilerParams(needs_layout_passes=False),
        out_type=jax.ShapeDtypeStruct((BATCH,), jnp.float32),
        scratch_types=[
            pltpu.VMEM((B_PER_W,), jnp.int32),
            pltpu.VMEM((B_PER_W,), jnp.int32),
            pltpu.VMEM((SLOTS * EPB, EMBED_DIM, LANES), jnp.float32),
            pltpu.VMEM((SLOTS * EPB, EMBED_DIM, LANES), jnp.float32),
            pltpu.VMEM((B_PER_W,), jnp.float32),
            pltpu.SemaphoreType.DMA,
            pltpu.SemaphoreType.DMA,
            pltpu.SemaphoreType.DMA,
        ],
    )
    return f(user, item, uemb_t, iemb_t)


# speedup vs baseline: 1.2320x; 1.0340x over previous
import jax
import jax.numpy as jnp
from jax import lax
from jax.experimental import pallas as pl
from jax.experimental.pallas import tpu as pltpu
from jax.experimental.pallas import tpu_sc as plsc

EMBED_DIM = 64
BATCH = 16384
N_ROWS = 1000001
LANES = 128

_info = plsc.get_sparse_core_info()
NC, NS, L = _info.num_cores, _info.num_subcores, _info.num_lanes
NW = NC * NS
B_PER_W = BATCH // NW
EPB = 2
SLOTS = 2


def _mf_kernel(user_hbm, item_hbm, uemb_hbm, iemb_hbm, out_hbm,
               idx_u, idx_i, buf_u, buf_i, out_v, sem):
    wid = lax.axis_index("s") * NC + lax.axis_index("c")
    base = wid * B_PER_W

    pltpu.sync_copy(user_hbm.at[pl.ds(base, B_PER_W)], idx_u)
    pltpu.sync_copy(item_hbm.at[pl.ds(base, B_PER_W)], idx_i)

    lane = lax.iota(jnp.int32, L)
    dvs = [j * L + lane for j in range(EMBED_DIM // L)]

    NSB = L // EPB
    NGROUPS = B_PER_W // L

    def window(emb, r):
        return emb.at[:, pl.ds((r // LANES) * LANES, LANES)]

    def issue(uvec, ivec, sb, slot):
        copies = []
        lanes_u = []
        lanes_i = []
        for e in range(EPB):
            ru = uvec[sb * EPB + e]
            ri = ivec[sb * EPB + e]
            copies.append(pltpu.async_copy(
                window(uemb_hbm, ru), buf_u.at[slot * EPB + e], sem))
            copies.append(pltpu.async_copy(
                window(iemb_hbm, ri), buf_i.at[slot * EPB + e], sem))
            lanes_u.append(jnp.full((L,), ru % LANES, jnp.int32))
            lanes_i.append(jnp.full((L,), ri % LANES, jnp.int32))
        return copies, lanes_u, lanes_i

    def group_body(g, carry):
        base_g = g * L
        uvec = idx_u[pl.ds(base_g, L)]
        ivec = idx_i[pl.ds(base_g, L)]
        res = jnp.zeros((L,), jnp.float32)
        pending = issue(uvec, ivec, 0, 0)
        for sb in range(NSB):
            slot = sb % SLOTS
            copies, lanes_u, lanes_i = pending
            if sb + 1 < NSB:
                pending = issue(uvec, ivec, sb + 1, (sb + 1) % SLOTS)
            for c in copies:
                c.wait()
            for e in range(EPB):
                acc = jnp.zeros((L,), jnp.float32)
                for j in range(EMBED_DIM // L):
                    gu = plsc.load_gather(
                        buf_u.at[slot * EPB + e], [dvs[j], lanes_u[e]])
                    gi = plsc.load_gather(
                        buf_i.at[slot * EPB + e], [dvs[j], lanes_i[e]])
                    acc = acc + gu * gi
                res = jnp.where(lane == sb * EPB + e, jnp.sum(acc), res)
        out_v[pl.ds(base_g, L)] = 1.0 / (1.0 + jnp.exp(-res))
        return carry

    lax.fori_loop(0, NGROUPS, group_body, 0)

    pltpu.sync_copy(out_v, out_hbm.at[pl.ds(base, B_PER_W)])


@jax.jit
def kernel(user, item, user_embed, item_embed):
    user = user.astype(jnp.int32)
    item = item.astype(jnp.int32)
    uemb_t = user_embed.T
    iemb_t = item_embed.T
    mesh = plsc.VectorSubcoreMesh(core_axis_name="c", subcore_axis_name="s")
    f = pl.kernel(
        _mf_kernel,
        mesh=mesh,
        compiler_params=pltpu.CompilerParams(needs_layout_passes=False),
        out_type=jax.ShapeDtypeStruct((BATCH,), jnp.float32),
        scratch_types=[
            pltpu.VMEM((B_PER_W,), jnp.int32),
            pltpu.VMEM((B_PER_W,), jnp.int32),
            pltpu.VMEM((SLOTS * EPB, EMBED_DIM, LANES), jnp.float32),
            pltpu.VMEM((SLOTS * EPB, EMBED_DIM, LANES), jnp.float32),
            pltpu.VMEM((B_PER_W,), jnp.float32),
            pltpu.SemaphoreType.DMA,
        ],
    )
    return f(user, item, uemb_t, iemb_t)
